# Initial kernel scaffold; baseline (speedup 1.0000x reference)
import functools

import jax
import jax.numpy as jnp
from jax import lax
from jax.experimental import pallas as pl
from jax.experimental.pallas import tpu as pltpu
from jax.experimental.pallas import tpu_sc as plsc

D = 32
NC, NS = 2, 16
NW = NC * NS
B = 16384 * 50
BPW = B // NW
GATHER = 128
NG = 10
K = GATHER * NG
CHUNKS = BPW // K

_mesh = plsc.VectorSubcoreMesh(core_axis_name="c", subcore_axis_name="s")


@functools.partial(
    pl.kernel,
    mesh=_mesh,
    out_type=jax.ShapeDtypeStruct((B, D), jnp.float32),
    scratch_types=[
        pltpu.VMEM((K,), jnp.int32),
        pltpu.VMEM((K, D), jnp.float32),
        pltpu.SemaphoreType.DMA,
    ],
)
def _emb_lookup(idx_hbm, table_hbm, out_hbm, idx_v, rows_v, sem):
    wid = lax.axis_index("s") * NC + lax.axis_index("c")
    base = wid * BPW

    def chunk_body(c, carry):
        off = base + c * K
        pltpu.sync_copy(idx_hbm.at[pl.ds(off, K)], idx_v)
        copies = [
            pltpu.async_copy(
                table_hbm.at[idx_v.at[pl.ds(j * GATHER, GATHER)]],
                rows_v.at[pl.ds(j * GATHER, GATHER)],
                sem,
            )
            for j in range(NG)
        ]
        for cp in copies:
            cp.wait()
        pltpu.sync_copy(rows_v, out_hbm.at[pl.ds(off, K)])
        return carry

    lax.fori_loop(0, CHUNKS, chunk_body, 0)


def kernel(x, atom_emb_weight):
    idx = x.reshape(-1).astype(jnp.int32)
    out = _emb_lookup(idx, atom_emb_weight)
    return out.reshape(x.shape + (D,))


# speedup vs baseline: 2.9623x; 2.9623x over previous
import functools

import jax
import jax.numpy as jnp
from jax import lax
from jax.experimental import pallas as pl
from jax.experimental.pallas import tpu as pltpu
from jax.experimental.pallas import tpu_sc as plsc

D = 32
NC, NS = 2, 16
NW = NC * NS
B = 16384 * 50
BPW = B // NW
GATHER = 128
NG = 10
K = GATHER * NG
CHUNKS = BPW // K

_mesh = plsc.VectorSubcoreMesh(core_axis_name="c", subcore_axis_name="s")


@functools.partial(
    pl.kernel,
    mesh=_mesh,
    out_type=jax.ShapeDtypeStruct((B, D), jnp.float32),
    scratch_types=[
        pltpu.VMEM((K,), jnp.int32),
        pltpu.VMEM((K, D), jnp.float32),
        pltpu.SemaphoreType.DMA,
    ],
    compiler_params=pltpu.CompilerParams(use_tc_tiling_on_sc=False),
)
def _emb_lookup(idx_hbm, table_hbm, out_hbm, idx_v, rows_v, sem):
    wid = lax.axis_index("s") * NC + lax.axis_index("c")
    base = wid * BPW

    def chunk_body(c, carry):
        off = base + c * K
        pltpu.sync_copy(idx_hbm.at[pl.ds(off, K)], idx_v)
        copies = [
            pltpu.async_copy(
                table_hbm.at[idx_v.at[pl.ds(j * GATHER, GATHER)]],
                rows_v.at[pl.ds(j * GATHER, GATHER)],
                sem,
            )
            for j in range(NG)
        ]
        for cp in copies:
            cp.wait()
        pltpu.sync_copy(rows_v, out_hbm.at[pl.ds(off, K)])
        return carry

    lax.fori_loop(0, CHUNKS, chunk_body, 0)


def kernel(x, atom_emb_weight):
    idx = x.reshape(-1).astype(jnp.int32)
    out = _emb_lookup(idx, atom_emb_weight)
    return out.reshape(x.shape + (D,))


# speedup vs baseline: 2.9659x; 1.0012x over previous
import functools

import jax
import jax.numpy as jnp
from jax import lax
from jax.experimental import pallas as pl
from jax.experimental.pallas import tpu as pltpu
from jax.experimental.pallas import tpu_sc as plsc

D = 32
NC, NS = 2, 16
NW = NC * NS
B = 16384 * 50
BPW = B // NW
GATHER = 1280
NG = 1
K = GATHER * NG
CHUNKS = BPW // K

_mesh = plsc.VectorSubcoreMesh(core_axis_name="c", subcore_axis_name="s")


@functools.partial(
    pl.kernel,
    mesh=_mesh,
    out_type=jax.ShapeDtypeStruct((B, D), jnp.float32),
    scratch_types=[
        pltpu.VMEM((K,), jnp.int32),
        pltpu.VMEM((K, D), jnp.float32),
        pltpu.SemaphoreType.DMA,
    ],
    compiler_params=pltpu.CompilerParams(use_tc_tiling_on_sc=False),
)
def _emb_lookup(idx_hbm, table_hbm, out_hbm, idx_v, rows_v, sem):
    wid = lax.axis_index("s") * NC + lax.axis_index("c")
    base = wid * BPW

    def chunk_body(c, carry):
        off = base + c * K
        pltpu.sync_copy(idx_hbm.at[pl.ds(off, K)], idx_v)
        copies = [
            pltpu.async_copy(
                table_hbm.at[idx_v.at[pl.ds(j * GATHER, GATHER)]],
                rows_v.at[pl.ds(j * GATHER, GATHER)],
                sem,
            )
            for j in range(NG)
        ]
        for cp in copies:
            cp.wait()
        pltpu.sync_copy(rows_v, out_hbm.at[pl.ds(off, K)])
        return carry

    lax.fori_loop(0, CHUNKS, chunk_body, 0)


def kernel(x, atom_emb_weight):
    idx = x.reshape(-1).astype(jnp.int32)
    out = _emb_lookup(idx, atom_emb_weight)
    return out.reshape(x.shape + (D,))


# speedup vs baseline: 6.1081x; 2.0594x over previous
import functools

import jax
import jax.numpy as jnp
from jax import lax
from jax.experimental import pallas as pl
from jax.experimental.pallas import tpu as pltpu
from jax.experimental.pallas import tpu_sc as plsc

D = 32
NC, NS = 2, 16
NW = NC * NS
B = 16384 * 50
BPW = B // NW
NB = 32
K = NB * 50
BATCH_PW = 16384 // NW
CHUNKS = BATCH_PW // NB

_mesh = plsc.VectorSubcoreMesh(core_axis_name="c", subcore_axis_name="s")


@functools.partial(
    pl.kernel,
    mesh=_mesh,
    out_type=jax.ShapeDtypeStruct((16384, 50, D), jnp.float32),
    scratch_types=[
        pltpu.VMEM((K,), jnp.int32),
        pltpu.VMEM((K, D), jnp.float32),
        pltpu.SemaphoreType.DMA,
        pltpu.SemaphoreType.DMA,
    ],
    compiler_params=pltpu.CompilerParams(use_tc_tiling_on_sc=False),
)
def _emb_lookup(idx_hbm, table_hbm, out_hbm, idx_v, rows_v, gsem, osem):
    wid = lax.axis_index("s") * NC + lax.axis_index("c")
    base = wid * BPW
    bbase = wid * BATCH_PW

    def chunk_body(c, carry):
        off = base + c * K
        pltpu.sync_copy(idx_hbm.at[pl.ds(off, K)], idx_v)
        pltpu.async_copy(table_hbm.at[idx_v], rows_v, gsem).wait()
        outs = [
            pltpu.async_copy(
                rows_v.at[pl.ds(i * 50, 50)],
                out_hbm.at[bbase + c * NB + i],
                osem,
            )
            for i in range(NB)
        ]
        for cp in outs:
            cp.wait()
        return carry

    lax.fori_loop(0, CHUNKS, chunk_body, 0)


def kernel(x, atom_emb_weight):
    return _emb_lookup(x.reshape(-1), atom_emb_weight)


# speedup vs baseline: 6.1454x; 1.0061x over previous
import functools

import jax
import jax.numpy as jnp
from jax import lax
from jax.experimental import pallas as pl
from jax.experimental.pallas import tpu as pltpu
from jax.experimental.pallas import tpu_sc as plsc

D = 32
HIST = 50
BATCH = 16384
NC, NS = 2, 16
NW = NC * NS
BPB = BATCH // NW

_mesh = plsc.VectorSubcoreMesh(core_axis_name="c", subcore_axis_name="s")


@functools.partial(
    pl.kernel,
    mesh=_mesh,
    out_type=jax.ShapeDtypeStruct((BATCH, HIST, D), jnp.float32),
    scratch_types=[
        pltpu.VMEM((BPB,), jnp.int32),
        pltpu.VMEM((BPB,), jnp.int32),
        pltpu.VMEM((BPB, D), jnp.float32),
        pltpu.VMEM((BPB, D), jnp.float32),
        pltpu.SemaphoreType.DMA,
        pltpu.SemaphoreType.DMA,
        pltpu.SemaphoreType.DMA,
        pltpu.SemaphoreType.DMA,
    ],
    compiler_params=pltpu.CompilerParams(use_tc_tiling_on_sc=False),
)
def _emb_lookup(xt_hbm, table_hbm, out_hbm, idx0, idx1, rows0, rows1,
                g0, g1, o0, o1):
    wid = lax.axis_index("s") * NC + lax.axis_index("c")
    b0 = wid * BPB
    idx = (idx0, idx1)
    rows = (rows0, rows1)
    gs = (g0, g1)
    os_ = (o0, o1)

    def load_and_gather(h, b):
        pltpu.sync_copy(xt_hbm.at[h, pl.ds(b0, BPB)], idx[b])
        pltpu.async_copy(table_hbm.at[idx[b]], rows[b], gs[b])

    def wait_gather(b):
        pltpu.make_async_copy(table_hbm.at[idx[b]], rows[b], gs[b]).wait()

    def write_out(h, b):
        pltpu.async_copy(rows[b], out_hbm.at[pl.ds(b0, BPB), h], os_[b])

    def wait_out(h, b):
        pltpu.make_async_copy(rows[b], out_hbm.at[pl.ds(b0, BPB), h],
                              os_[b]).wait()

    load_and_gather(0, 0)
    load_and_gather(1, 1)

    def body(t, carry):
        for b in range(2):
            h = 2 * t + b
            wait_gather(b)
            write_out(h, b)
            wait_out(h, b)
            load_and_gather(h + 2, b)
        return carry

    lax.fori_loop(0, HIST // 2 - 1, body, 0)
    for b in range(2):
        h = HIST - 2 + b
        wait_gather(b)
        write_out(h, b)
        wait_out(h, b)


def kernel(x, atom_emb_weight):
    return _emb_lookup(x.T, atom_emb_weight)
